# baseline (speedup 1.0000x reference)
import functools

import jax
import jax.numpy as jnp
from jax import lax
from jax.experimental import pallas as pl
from jax.experimental.pallas import tpu as pltpu
from jax.experimental.pallas import tpu_sc as plsc

N_NODES = 10000
D = 128
N_GRAPHS = 16

NPAD = 10240
N_ZPAD = NPAD - N_NODES
NC = 2
NS = 16
NW = NC * NS
CHUNK = 128
CPT = 80
PHASES = 2
CPP = CPT // PHASES
NBUF = 2
EPAD = NW * CPT * CHUNK
ROWS_PT = NPAD // NS

_sc_mesh = plsc.VectorSubcoreMesh(core_axis_name="c", subcore_axis_name="s")


def _sc_degree_body(dst_hbm, out_hbm, dst_v, ones_v, zero_v, deg_acc):
    c = lax.axis_index("c")
    s = lax.axis_index("s")
    w = c * NS + s

    for i in range(CHUNK // 16):
        ones_v[pl.ds(i * 16, 16)] = jnp.ones((16,), jnp.float32)
        zero_v[pl.ds(i * 16, 16)] = jnp.zeros((16,), jnp.float32)

    for t in range(ROWS_PT // CHUNK):
        pltpu.sync_copy(zero_v, deg_acc.at[pl.ds(s * ROWS_PT + t * CHUNK, CHUNK)])

    pltpu.sync_copy(dst_hbm.at[w], dst_v)

    plsc.subcore_barrier()

    def body(j, carry):
        pltpu.sync_copy(ones_v, deg_acc.at[dst_v.at[j]], add=True)
        return carry

    lax.fori_loop(0, CPT, body, 0, unroll=4)

    plsc.subcore_barrier()
    pltpu.sync_copy(deg_acc.at[pl.ds(s * ROWS_PT, ROWS_PT)],
                    out_hbm.at[c, pl.ds(s * ROWS_PT, ROWS_PT)])


_sc_degree = pl.kernel(
    _sc_degree_body,
    out_type=jax.ShapeDtypeStruct((NC, NPAD), jnp.float32),
    mesh=_sc_mesh,
    scratch_types=[
        pltpu.VMEM((CPT, CHUNK), jnp.int32),
        pltpu.VMEM((CHUNK,), jnp.float32),
        pltpu.VMEM((CHUNK,), jnp.float32),
        pltpu.VMEM_SHARED((NPAD,), jnp.float32),
    ],
)


def _sc_edges_body_full(p_hbm, src_hbm, dst_hbm, out_hbm,
                        src_v, dst_v, rows, acc, gsems):
    c = lax.axis_index("c")
    s = lax.axis_index("s")
    w = c * NS + s

    pltpu.sync_copy(p_hbm.at[pl.ds(s * ROWS_PT, ROWS_PT)],
                    acc.at[pl.ds(s * ROWS_PT, ROWS_PT)])

    def gather_start(j, b):
        pltpu.async_copy(p_hbm.at[src_v.at[j]], rows.at[b], gsems.at[b])

    def gather_wait(j, b):
        pltpu.make_async_copy(p_hbm.at[src_v.at[j]], rows.at[b],
                              gsems.at[b]).wait()

    def scatter(j, b):
        pltpu.sync_copy(rows.at[b], acc.at[dst_v.at[j]], add=True)

    for ph in range(PHASES):
        pltpu.sync_copy(src_hbm.at[w, pl.ds(ph * CPP, CPP)], src_v)
        pltpu.sync_copy(dst_hbm.at[w, pl.ds(ph * CPP, CPP)], dst_v)
        gather_start(0, 0)
        if ph == 0:
            plsc.subcore_barrier()

        def body(i, carry):
            j0 = 2 * i
            j1 = 2 * i + 1
            gather_start(j1, 1)
            gather_wait(j0, 0)
            scatter(j0, 0)

            @pl.when(j1 + 1 < CPP)
            def _():
                gather_start(j1 + 1, 0)

            gather_wait(j1, 1)
            scatter(j1, 1)
            return carry

        lax.fori_loop(0, CPP // 2, body, 0)

    plsc.subcore_barrier()
    pltpu.sync_copy(acc.at[pl.ds(s * ROWS_PT, ROWS_PT)],
                    out_hbm.at[c, pl.ds(s * ROWS_PT, ROWS_PT)])


_sc_edges = pl.kernel(
    _sc_edges_body_full,
    out_type=jax.ShapeDtypeStruct((NC, NPAD, D), jnp.float32),
    mesh=_sc_mesh,
    scratch_types=[
        pltpu.VMEM((CPP, CHUNK), jnp.int32),
        pltpu.VMEM((CPP, CHUNK), jnp.int32),
        pltpu.VMEM((NBUF, CHUNK, D), jnp.float32),
        pltpu.VMEM_SHARED((NPAD, D), jnp.float32),
        pltpu.SemaphoreType.DMA((NBUF,)),
    ],
)


BLK = 5120
GRID = NPAD // BLK


def _tc_first_body(x_ref, w_ref, d0_ref, d1_ref, p_ref, dis_ref):
    deg = d0_ref[...] + d1_ref[...] + 1.0
    dis = lax.rsqrt(deg)
    dis_ref[...] = dis
    p_ref[...] = jnp.dot(x_ref[...], w_ref[...],
                         preferred_element_type=jnp.float32) * dis


def _tc_first(x_pad, W1, d0, d1):
    return pl.pallas_call(
        _tc_first_body,
        grid=(GRID,),
        in_specs=[
            pl.BlockSpec((BLK, D), lambda i: (i, 0)),
            pl.BlockSpec((D, D), lambda i: (0, 0)),
            pl.BlockSpec((BLK, 1), lambda i: (i, 0)),
            pl.BlockSpec((BLK, 1), lambda i: (i, 0)),
        ],
        out_specs=[
            pl.BlockSpec((BLK, D), lambda i: (i, 0)),
            pl.BlockSpec((BLK, 1), lambda i: (i, 0)),
        ],
        out_shape=[
            jax.ShapeDtypeStruct((NPAD, D), jnp.float32),
            jax.ShapeDtypeStruct((NPAD, 1), jnp.float32),
        ],
    )(x_pad, W1, d0, d1)


def _tc_mid_body(a0_ref, a1_ref, p_ref, dis_ref, b1_ref, w2_ref, out_ref, *, blk):
    i = pl.program_id(0)
    agg = a0_ref[...] + a1_ref[...] - p_ref[...]
    h = jnp.maximum(agg * dis_ref[...] + b1_ref[...], 0.0)
    p2 = jnp.dot(h, w2_ref[...], preferred_element_type=jnp.float32) * dis_ref[...]
    row = i * blk + lax.broadcasted_iota(jnp.int32, (blk, 1), 0)
    out_ref[...] = jnp.where(row < N_NODES, p2, 0.0)


def _tc_mid(a0, a1, p1, dis, b1, W2):
    return pl.pallas_call(
        functools.partial(_tc_mid_body, blk=BLK),
        grid=(GRID,),
        in_specs=[
            pl.BlockSpec((BLK, D), lambda i: (i, 0)),
            pl.BlockSpec((BLK, D), lambda i: (i, 0)),
            pl.BlockSpec((BLK, D), lambda i: (i, 0)),
            pl.BlockSpec((BLK, 1), lambda i: (i, 0)),
            pl.BlockSpec((1, D), lambda i: (0, 0)),
            pl.BlockSpec((D, D), lambda i: (0, 0)),
        ],
        out_specs=pl.BlockSpec((BLK, D), lambda i: (i, 0)),
        out_shape=jax.ShapeDtypeStruct((NPAD, D), jnp.float32),
    )(a0, a1, p1, dis, b1, W2)


def _tc_pool_body(a0_ref, a1_ref, p_ref, dis_ref, b2_ref, batch_ref,
                  wl_ref, bl_ref, out_ref, gsum, cnt, *, blk, grid):
    i = pl.program_id(0)

    @pl.when(i == 0)
    def _():
        gsum[...] = jnp.zeros_like(gsum)
        cnt[...] = jnp.zeros_like(cnt)

    agg = a0_ref[...] + a1_ref[...] - p_ref[...]
    h2 = agg * dis_ref[...] + b2_ref[...]
    gids = lax.broadcasted_iota(jnp.int32, (1, N_GRAPHS), 1)
    oh = (batch_ref[...] == gids).astype(jnp.float32)
    dnums = (((0,), (0,)), ((), ()))
    gsum[...] += lax.dot_general(oh, h2, dnums,
                                 preferred_element_type=jnp.float32)
    cnt[...] += lax.dot_general(oh, jnp.ones((blk, D), jnp.float32), dnums,
                                preferred_element_type=jnp.float32)

    @pl.when(i == grid - 1)
    def _():
        g = gsum[...] / jnp.maximum(cnt[...], 1.0)
        out_ref[...] = jnp.dot(g, wl_ref[...],
                               preferred_element_type=jnp.float32) + bl_ref[...]


def _tc_pool(a0, a1, p2, dis, b2, batch_col, Wl, bl):
    return pl.pallas_call(
        functools.partial(_tc_pool_body, blk=BLK, grid=GRID),
        grid=(GRID,),
        in_specs=[
            pl.BlockSpec((BLK, D), lambda i: (i, 0)),
            pl.BlockSpec((BLK, D), lambda i: (i, 0)),
            pl.BlockSpec((BLK, D), lambda i: (i, 0)),
            pl.BlockSpec((BLK, 1), lambda i: (i, 0)),
            pl.BlockSpec((1, D), lambda i: (0, 0)),
            pl.BlockSpec((BLK, 1), lambda i: (i, 0)),
            pl.BlockSpec((D, D), lambda i: (0, 0)),
            pl.BlockSpec((1, D), lambda i: (0, 0)),
        ],
        out_specs=pl.BlockSpec((N_GRAPHS, D), lambda i: (0, 0)),
        out_shape=jax.ShapeDtypeStruct((N_GRAPHS, D), jnp.float32),
        scratch_shapes=[
            pltpu.VMEM((N_GRAPHS, D), jnp.float32),
            pltpu.VMEM((N_GRAPHS, D), jnp.float32),
        ],
    )(a0, a1, p2, dis, b2, batch_col, Wl, bl)


@jax.jit
def kernel(x, edge_index, batch, W1, b1, W2, b2, Wl, bl):
    x_pad = jnp.pad(x, ((0, N_ZPAD), (0, 0)))
    src = edge_index[0].astype(jnp.int32)
    dst = edge_index[1].astype(jnp.int32)
    n_epad = EPAD - src.shape[0]
    pad_idx = N_NODES + (jnp.arange(n_epad, dtype=jnp.int32) % N_ZPAD)
    src_arr = jnp.concatenate([src, pad_idx]).reshape(NW, CPT, CHUNK)
    dst_arr = jnp.concatenate([dst, pad_idx]).reshape(NW, CPT, CHUNK)
    batch_col = jnp.concatenate(
        [batch.astype(jnp.int32),
         jnp.full((N_ZPAD,), N_GRAPHS, jnp.int32)]).reshape(NPAD, 1)
    b1c = b1.reshape(1, D)
    b2c = b2.reshape(1, D)
    blc = bl.reshape(1, D)

    deg_parts = _sc_degree(dst_arr)
    d0 = deg_parts[0].reshape(NPAD, 1)
    d1 = deg_parts[1].reshape(NPAD, 1)
    p1, dis = _tc_first(x_pad, W1, d0, d1)

    acc1 = _sc_edges(p1, src_arr, dst_arr)
    p2 = _tc_mid(acc1[0], acc1[1], p1, dis, b1c, W2)

    acc2 = _sc_edges(p2, src_arr, dst_arr)
    return _tc_pool(acc2[0], acc2[1], p2, dis, b2c, batch_col, Wl, blc)


# speedup vs baseline: 1.0036x; 1.0036x over previous
import functools

import jax
import jax.numpy as jnp
from jax import lax
from jax.experimental import pallas as pl
from jax.experimental.pallas import tpu as pltpu
from jax.experimental.pallas import tpu_sc as plsc

N_NODES = 10000
D = 128
N_GRAPHS = 16

NPAD = 10240
N_ZPAD = NPAD - N_NODES
NC = 2
NS = 16
NW = NC * NS
CHUNK = 128
CPT = 80
PHASES = 2
CPP = CPT // PHASES
NBUF = 2
EPAD = NW * CPT * CHUNK
ROWS_PT = NPAD // NS

_sc_mesh = plsc.VectorSubcoreMesh(core_axis_name="c", subcore_axis_name="s")


def _sc_degree_body(dst_hbm, out_hbm, dst_v, ones_v, zero_v, deg_acc):
    c = lax.axis_index("c")
    s = lax.axis_index("s")
    w = c * NS + s

    for i in range(CHUNK // 16):
        ones_v[pl.ds(i * 16, 16)] = jnp.ones((16,), jnp.float32)
        zero_v[pl.ds(i * 16, 16)] = jnp.zeros((16,), jnp.float32)

    for t in range(ROWS_PT // CHUNK):
        pltpu.sync_copy(zero_v, deg_acc.at[pl.ds(s * ROWS_PT + t * CHUNK, CHUNK)])

    pltpu.sync_copy(dst_hbm.at[w], dst_v)

    plsc.subcore_barrier()

    def body(j, carry):
        pltpu.sync_copy(ones_v, deg_acc.at[dst_v.at[j]], add=True)
        return carry

    lax.fori_loop(0, CPT, body, 0, unroll=4)

    plsc.subcore_barrier()
    pltpu.sync_copy(deg_acc.at[pl.ds(s * ROWS_PT, ROWS_PT)],
                    out_hbm.at[c, pl.ds(s * ROWS_PT, ROWS_PT)])


_sc_degree = pl.kernel(
    _sc_degree_body,
    out_type=jax.ShapeDtypeStruct((NC, NPAD), jnp.float32),
    mesh=_sc_mesh,
    scratch_types=[
        pltpu.VMEM((CPT, CHUNK), jnp.int32),
        pltpu.VMEM((CHUNK,), jnp.float32),
        pltpu.VMEM((CHUNK,), jnp.float32),
        pltpu.VMEM_SHARED((NPAD,), jnp.float32),
    ],
)


def _sc_edges_body_full(p_hbm, src_hbm, dst_hbm, out_hbm,
                        src_v, dst_v, rows, acc, gsems):
    c = lax.axis_index("c")
    s = lax.axis_index("s")
    w = c * NS + s

    pltpu.sync_copy(p_hbm.at[pl.ds(s * ROWS_PT, ROWS_PT)],
                    acc.at[pl.ds(s * ROWS_PT, ROWS_PT)])

    def gather_start(j, b):
        pltpu.async_copy(p_hbm.at[src_v.at[j]], rows.at[b], gsems.at[b])

    def gather_wait(j, b):
        pltpu.make_async_copy(p_hbm.at[src_v.at[j]], rows.at[b],
                              gsems.at[b]).wait()

    def scatter(j, b):
        pltpu.sync_copy(rows.at[b], acc.at[dst_v.at[j]], add=True)

    for ph in range(PHASES):
        pltpu.sync_copy(src_hbm.at[w, pl.ds(ph * CPP, CPP)], src_v)
        pltpu.sync_copy(dst_hbm.at[w, pl.ds(ph * CPP, CPP)], dst_v)
        gather_start(0, 0)
        if ph == 0:
            plsc.subcore_barrier()

        def body(i, carry):
            j0 = 2 * i
            j1 = 2 * i + 1
            gather_start(j1, 1)
            gather_wait(j0, 0)
            scatter(j0, 0)

            @pl.when(j1 + 1 < CPP)
            def _():
                gather_start(j1 + 1, 0)

            gather_wait(j1, 1)
            scatter(j1, 1)
            return carry

        lax.fori_loop(0, CPP // 2, body, 0)

    plsc.subcore_barrier()
    pltpu.sync_copy(acc.at[pl.ds(s * ROWS_PT, ROWS_PT)],
                    out_hbm.at[c, pl.ds(s * ROWS_PT, ROWS_PT)])


_sc_edges = pl.kernel(
    _sc_edges_body_full,
    out_type=jax.ShapeDtypeStruct((NC, NPAD, D), jnp.float32),
    mesh=_sc_mesh,
    scratch_types=[
        pltpu.VMEM((CPP, CHUNK), jnp.int32),
        pltpu.VMEM((CPP, CHUNK), jnp.int32),
        pltpu.VMEM((NBUF, CHUNK, D), jnp.float32),
        pltpu.VMEM_SHARED((NPAD, D), jnp.float32),
        pltpu.SemaphoreType.DMA((NBUF,)),
    ],
)


BLK = 5120
GRID = NPAD // BLK


def _tc_matmul_body(x_ref, w_ref, h_ref):
    h_ref[...] = jnp.dot(x_ref[...], w_ref[...],
                         preferred_element_type=jnp.float32)


def _tc_matmul(x_pad, W1):
    return pl.pallas_call(
        _tc_matmul_body,
        grid=(GRID,),
        in_specs=[
            pl.BlockSpec((BLK, D), lambda i: (i, 0)),
            pl.BlockSpec((D, D), lambda i: (0, 0)),
        ],
        out_specs=pl.BlockSpec((BLK, D), lambda i: (i, 0)),
        out_shape=jax.ShapeDtypeStruct((NPAD, D), jnp.float32),
    )(x_pad, W1)


def _tc_scale_body(h_ref, d0_ref, d1_ref, p_ref, dis_ref):
    deg = d0_ref[...] + d1_ref[...] + 1.0
    dis = lax.rsqrt(deg)
    dis_ref[...] = dis
    p_ref[...] = h_ref[...] * dis


def _tc_scale(h1, d0, d1):
    return pl.pallas_call(
        _tc_scale_body,
        grid=(GRID,),
        in_specs=[
            pl.BlockSpec((BLK, D), lambda i: (i, 0)),
            pl.BlockSpec((BLK, 1), lambda i: (i, 0)),
            pl.BlockSpec((BLK, 1), lambda i: (i, 0)),
        ],
        out_specs=[
            pl.BlockSpec((BLK, D), lambda i: (i, 0)),
            pl.BlockSpec((BLK, 1), lambda i: (i, 0)),
        ],
        out_shape=[
            jax.ShapeDtypeStruct((NPAD, D), jnp.float32),
            jax.ShapeDtypeStruct((NPAD, 1), jnp.float32),
        ],
    )(h1, d0, d1)


def _tc_mid_body(a0_ref, a1_ref, p_ref, dis_ref, b1_ref, w2_ref, out_ref, *, blk):
    i = pl.program_id(0)
    agg = a0_ref[...] + a1_ref[...] - p_ref[...]
    h = jnp.maximum(agg * dis_ref[...] + b1_ref[...], 0.0)
    p2 = jnp.dot(h, w2_ref[...], preferred_element_type=jnp.float32) * dis_ref[...]
    row = i * blk + lax.broadcasted_iota(jnp.int32, (blk, 1), 0)
    out_ref[...] = jnp.where(row < N_NODES, p2, 0.0)


def _tc_mid(a0, a1, p1, dis, b1, W2):
    return pl.pallas_call(
        functools.partial(_tc_mid_body, blk=BLK),
        grid=(GRID,),
        in_specs=[
            pl.BlockSpec((BLK, D), lambda i: (i, 0)),
            pl.BlockSpec((BLK, D), lambda i: (i, 0)),
            pl.BlockSpec((BLK, D), lambda i: (i, 0)),
            pl.BlockSpec((BLK, 1), lambda i: (i, 0)),
            pl.BlockSpec((1, D), lambda i: (0, 0)),
            pl.BlockSpec((D, D), lambda i: (0, 0)),
        ],
        out_specs=pl.BlockSpec((BLK, D), lambda i: (i, 0)),
        out_shape=jax.ShapeDtypeStruct((NPAD, D), jnp.float32),
    )(a0, a1, p1, dis, b1, W2)


def _tc_pool_body(a0_ref, a1_ref, p_ref, dis_ref, b2_ref, batch_ref,
                  wl_ref, bl_ref, out_ref, gsum, cnt, *, blk, grid):
    i = pl.program_id(0)

    @pl.when(i == 0)
    def _():
        gsum[...] = jnp.zeros_like(gsum)
        cnt[...] = jnp.zeros_like(cnt)

    agg = a0_ref[...] + a1_ref[...] - p_ref[...]
    h2 = agg * dis_ref[...] + b2_ref[...]
    gids = lax.broadcasted_iota(jnp.int32, (1, N_GRAPHS), 1)
    oh = (batch_ref[...] == gids).astype(jnp.float32)
    dnums = (((0,), (0,)), ((), ()))
    gsum[...] += lax.dot_general(oh, h2, dnums,
                                 preferred_element_type=jnp.float32)
    cnt[...] += lax.dot_general(oh, jnp.ones((blk, D), jnp.float32), dnums,
                                preferred_element_type=jnp.float32)

    @pl.when(i == grid - 1)
    def _():
        g = gsum[...] / jnp.maximum(cnt[...], 1.0)
        out_ref[...] = jnp.dot(g, wl_ref[...],
                               preferred_element_type=jnp.float32) + bl_ref[...]


def _tc_pool(a0, a1, p2, dis, b2, batch_col, Wl, bl):
    return pl.pallas_call(
        functools.partial(_tc_pool_body, blk=BLK, grid=GRID),
        grid=(GRID,),
        in_specs=[
            pl.BlockSpec((BLK, D), lambda i: (i, 0)),
            pl.BlockSpec((BLK, D), lambda i: (i, 0)),
            pl.BlockSpec((BLK, D), lambda i: (i, 0)),
            pl.BlockSpec((BLK, 1), lambda i: (i, 0)),
            pl.BlockSpec((1, D), lambda i: (0, 0)),
            pl.BlockSpec((BLK, 1), lambda i: (i, 0)),
            pl.BlockSpec((D, D), lambda i: (0, 0)),
            pl.BlockSpec((1, D), lambda i: (0, 0)),
        ],
        out_specs=pl.BlockSpec((N_GRAPHS, D), lambda i: (0, 0)),
        out_shape=jax.ShapeDtypeStruct((N_GRAPHS, D), jnp.float32),
        scratch_shapes=[
            pltpu.VMEM((N_GRAPHS, D), jnp.float32),
            pltpu.VMEM((N_GRAPHS, D), jnp.float32),
        ],
    )(a0, a1, p2, dis, b2, batch_col, Wl, bl)


@jax.jit
def kernel(x, edge_index, batch, W1, b1, W2, b2, Wl, bl):
    x_pad = jnp.pad(x, ((0, N_ZPAD), (0, 0)))
    src = edge_index[0].astype(jnp.int32)
    dst = edge_index[1].astype(jnp.int32)
    n_epad = EPAD - src.shape[0]
    pad_idx = N_NODES + (jnp.arange(n_epad, dtype=jnp.int32) % N_ZPAD)
    src_arr = jnp.concatenate([src, pad_idx]).reshape(NW, CPT, CHUNK)
    dst_arr = jnp.concatenate([dst, pad_idx]).reshape(NW, CPT, CHUNK)
    batch_col = jnp.concatenate(
        [batch.astype(jnp.int32),
         jnp.full((N_ZPAD,), N_GRAPHS, jnp.int32)]).reshape(NPAD, 1)
    b1c = b1.reshape(1, D)
    b2c = b2.reshape(1, D)
    blc = bl.reshape(1, D)

    deg_parts = _sc_degree(dst_arr)
    h1 = _tc_matmul(x_pad, W1)
    d0 = deg_parts[0].reshape(NPAD, 1)
    d1 = deg_parts[1].reshape(NPAD, 1)
    p1, dis = _tc_scale(h1, d0, d1)

    acc1 = _sc_edges(p1, src_arr, dst_arr)
    p2 = _tc_mid(acc1[0], acc1[1], p1, dis, b1c, W2)

    acc2 = _sc_edges(p2, src_arr, dst_arr)
    return _tc_pool(acc2[0], acc2[1], p2, dis, b2c, batch_col, Wl, blc)


# speedup vs baseline: 1.0194x; 1.0157x over previous
import functools

import jax
import jax.numpy as jnp
from jax import lax
from jax.experimental import pallas as pl
from jax.experimental.pallas import tpu as pltpu
from jax.experimental.pallas import tpu_sc as plsc

N_NODES = 10000
D = 128
N_GRAPHS = 16

NPAD = 10240
N_ZPAD = NPAD - N_NODES
NC = 2
NS = 16
NW = NC * NS
CHUNK = 128
CPT = 80
PHASES = 2
CPP = CPT // PHASES
NBUF = 2
EPAD = NW * CPT * CHUNK
ROWS_PT = NPAD // NS

_sc_mesh = plsc.VectorSubcoreMesh(core_axis_name="c", subcore_axis_name="s")


def _sc_degree_body(dst_hbm, out_hbm, dst_v, ones_v, zero_v, deg_acc):
    c = lax.axis_index("c")
    s = lax.axis_index("s")
    w = c * NS + s

    for i in range(CHUNK // 16):
        ones_v[pl.ds(i * 16, 16)] = jnp.ones((16,), jnp.float32)
        zero_v[pl.ds(i * 16, 16)] = jnp.zeros((16,), jnp.float32)

    for t in range(ROWS_PT // CHUNK):
        pltpu.sync_copy(zero_v, deg_acc.at[pl.ds(s * ROWS_PT + t * CHUNK, CHUNK)])

    pltpu.sync_copy(dst_hbm.at[w], dst_v)

    plsc.subcore_barrier()

    def body(j, carry):
        pltpu.sync_copy(ones_v, deg_acc.at[dst_v.at[j]], add=True)
        return carry

    lax.fori_loop(0, CPT, body, 0, unroll=4)

    plsc.subcore_barrier()
    pltpu.sync_copy(deg_acc.at[pl.ds(s * ROWS_PT, ROWS_PT)],
                    out_hbm.at[c, pl.ds(s * ROWS_PT, ROWS_PT)])


_sc_degree = pl.kernel(
    _sc_degree_body,
    out_type=jax.ShapeDtypeStruct((NC, NPAD), jnp.float32),
    mesh=_sc_mesh,
    scratch_types=[
        pltpu.VMEM((CPT, CHUNK), jnp.int32),
        pltpu.VMEM((CHUNK,), jnp.float32),
        pltpu.VMEM((CHUNK,), jnp.float32),
        pltpu.VMEM_SHARED((NPAD,), jnp.float32),
    ],
)


def _sc_edges_body_full(p_hbm, src_hbm, dst_hbm, out_hbm,
                        src_v, dst_v, rows, acc, gsems, isem):
    c = lax.axis_index("c")
    s = lax.axis_index("s")
    w = c * NS + s

    init = pltpu.async_copy(p_hbm.at[pl.ds(s * ROWS_PT, ROWS_PT)],
                            acc.at[pl.ds(s * ROWS_PT, ROWS_PT)], isem)

    def gather_start(j, b):
        pltpu.async_copy(p_hbm.at[src_v.at[j]], rows.at[b], gsems.at[b])

    def gather_wait(j, b):
        pltpu.make_async_copy(p_hbm.at[src_v.at[j]], rows.at[b],
                              gsems.at[b]).wait()

    def scatter(j, b):
        pltpu.sync_copy(rows.at[b], acc.at[dst_v.at[j]], add=True)

    for ph in range(PHASES):
        pltpu.sync_copy(src_hbm.at[w, pl.ds(ph * CPP, CPP)], src_v)
        pltpu.sync_copy(dst_hbm.at[w, pl.ds(ph * CPP, CPP)], dst_v)
        gather_start(0, 0)
        if ph == 0:
            init.wait()
            plsc.subcore_barrier()

        def body(i, carry):
            j0 = 2 * i
            j1 = 2 * i + 1
            gather_start(j1, 1)
            gather_wait(j0, 0)
            scatter(j0, 0)

            @pl.when(j1 + 1 < CPP)
            def _():
                gather_start(j1 + 1, 0)

            gather_wait(j1, 1)
            scatter(j1, 1)
            return carry

        lax.fori_loop(0, CPP // 2, body, 0)

    plsc.subcore_barrier()
    pltpu.sync_copy(acc.at[pl.ds(s * ROWS_PT, ROWS_PT)],
                    out_hbm.at[c, pl.ds(s * ROWS_PT, ROWS_PT)])


_sc_edges = pl.kernel(
    _sc_edges_body_full,
    out_type=jax.ShapeDtypeStruct((NC, NPAD, D), jnp.float32),
    mesh=_sc_mesh,
    scratch_types=[
        pltpu.VMEM((CPP, CHUNK), jnp.int32),
        pltpu.VMEM((CPP, CHUNK), jnp.int32),
        pltpu.VMEM((NBUF, CHUNK, D), jnp.float32),
        pltpu.VMEM_SHARED((NPAD, D), jnp.float32),
        pltpu.SemaphoreType.DMA((NBUF,)),
        pltpu.SemaphoreType.DMA,
    ],
)


BLK = 5120
GRID = NPAD // BLK


def _tc_matmul_body(x_ref, w_ref, h_ref):
    h_ref[...] = jnp.dot(x_ref[...], w_ref[...],
                         preferred_element_type=jnp.float32)


def _tc_matmul(x_pad, W1):
    return pl.pallas_call(
        _tc_matmul_body,
        grid=(GRID,),
        in_specs=[
            pl.BlockSpec((BLK, D), lambda i: (i, 0)),
            pl.BlockSpec((D, D), lambda i: (0, 0)),
        ],
        out_specs=pl.BlockSpec((BLK, D), lambda i: (i, 0)),
        out_shape=jax.ShapeDtypeStruct((NPAD, D), jnp.float32),
    )(x_pad, W1)


def _tc_scale_body(h_ref, d0_ref, d1_ref, p_ref, dis_ref):
    deg = d0_ref[...] + d1_ref[...] + 1.0
    dis = lax.rsqrt(deg)
    dis_ref[...] = dis
    p_ref[...] = h_ref[...] * dis


def _tc_scale(h1, d0, d1):
    return pl.pallas_call(
        _tc_scale_body,
        grid=(GRID,),
        in_specs=[
            pl.BlockSpec((BLK, D), lambda i: (i, 0)),
            pl.BlockSpec((BLK, 1), lambda i: (i, 0)),
            pl.BlockSpec((BLK, 1), lambda i: (i, 0)),
        ],
        out_specs=[
            pl.BlockSpec((BLK, D), lambda i: (i, 0)),
            pl.BlockSpec((BLK, 1), lambda i: (i, 0)),
        ],
        out_shape=[
            jax.ShapeDtypeStruct((NPAD, D), jnp.float32),
            jax.ShapeDtypeStruct((NPAD, 1), jnp.float32),
        ],
    )(h1, d0, d1)


def _tc_mid_body(a0_ref, a1_ref, p_ref, dis_ref, b1_ref, w2_ref, out_ref, *, blk):
    i = pl.program_id(0)
    agg = a0_ref[...] + a1_ref[...] - p_ref[...]
    h = jnp.maximum(agg * dis_ref[...] + b1_ref[...], 0.0)
    p2 = jnp.dot(h, w2_ref[...], preferred_element_type=jnp.float32) * dis_ref[...]
    row = i * blk + lax.broadcasted_iota(jnp.int32, (blk, 1), 0)
    out_ref[...] = jnp.where(row < N_NODES, p2, 0.0)


def _tc_mid(a0, a1, p1, dis, b1, W2):
    return pl.pallas_call(
        functools.partial(_tc_mid_body, blk=BLK),
        grid=(GRID,),
        in_specs=[
            pl.BlockSpec((BLK, D), lambda i: (i, 0)),
            pl.BlockSpec((BLK, D), lambda i: (i, 0)),
            pl.BlockSpec((BLK, D), lambda i: (i, 0)),
            pl.BlockSpec((BLK, 1), lambda i: (i, 0)),
            pl.BlockSpec((1, D), lambda i: (0, 0)),
            pl.BlockSpec((D, D), lambda i: (0, 0)),
        ],
        out_specs=pl.BlockSpec((BLK, D), lambda i: (i, 0)),
        out_shape=jax.ShapeDtypeStruct((NPAD, D), jnp.float32),
    )(a0, a1, p1, dis, b1, W2)


def _tc_pool_body(a0_ref, a1_ref, p_ref, dis_ref, b2_ref, batch_ref,
                  wl_ref, bl_ref, out_ref, gsum, cnt, *, blk, grid):
    i = pl.program_id(0)

    @pl.when(i == 0)
    def _():
        gsum[...] = jnp.zeros_like(gsum)
        cnt[...] = jnp.zeros_like(cnt)

    agg = a0_ref[...] + a1_ref[...] - p_ref[...]
    h2 = agg * dis_ref[...] + b2_ref[...]
    gids = lax.broadcasted_iota(jnp.int32, (1, N_GRAPHS), 1)
    oh = (batch_ref[...] == gids).astype(jnp.float32)
    dnums = (((0,), (0,)), ((), ()))
    gsum[...] += lax.dot_general(oh, h2, dnums,
                                 preferred_element_type=jnp.float32)
    cnt[...] += lax.dot_general(oh, jnp.ones((blk, D), jnp.float32), dnums,
                                preferred_element_type=jnp.float32)

    @pl.when(i == grid - 1)
    def _():
        g = gsum[...] / jnp.maximum(cnt[...], 1.0)
        out_ref[...] = jnp.dot(g, wl_ref[...],
                               preferred_element_type=jnp.float32) + bl_ref[...]


def _tc_pool(a0, a1, p2, dis, b2, batch_col, Wl, bl):
    return pl.pallas_call(
        functools.partial(_tc_pool_body, blk=BLK, grid=GRID),
        grid=(GRID,),
        in_specs=[
            pl.BlockSpec((BLK, D), lambda i: (i, 0)),
            pl.BlockSpec((BLK, D), lambda i: (i, 0)),
            pl.BlockSpec((BLK, D), lambda i: (i, 0)),
            pl.BlockSpec((BLK, 1), lambda i: (i, 0)),
            pl.BlockSpec((1, D), lambda i: (0, 0)),
            pl.BlockSpec((BLK, 1), lambda i: (i, 0)),
            pl.BlockSpec((D, D), lambda i: (0, 0)),
            pl.BlockSpec((1, D), lambda i: (0, 0)),
        ],
        out_specs=pl.BlockSpec((N_GRAPHS, D), lambda i: (0, 0)),
        out_shape=jax.ShapeDtypeStruct((N_GRAPHS, D), jnp.float32),
        scratch_shapes=[
            pltpu.VMEM((N_GRAPHS, D), jnp.float32),
            pltpu.VMEM((N_GRAPHS, D), jnp.float32),
        ],
    )(a0, a1, p2, dis, b2, batch_col, Wl, bl)


@jax.jit
def kernel(x, edge_index, batch, W1, b1, W2, b2, Wl, bl):
    x_pad = jnp.pad(x, ((0, N_ZPAD), (0, 0)))
    src = edge_index[0].astype(jnp.int32)
    dst = edge_index[1].astype(jnp.int32)
    n_epad = EPAD - src.shape[0]
    pad_idx = N_NODES + (jnp.arange(n_epad, dtype=jnp.int32) % N_ZPAD)
    src_arr = jnp.concatenate([src, pad_idx]).reshape(NW, CPT, CHUNK)
    dst_arr = jnp.concatenate([dst, pad_idx]).reshape(NW, CPT, CHUNK)
    batch_col = jnp.concatenate(
        [batch.astype(jnp.int32),
         jnp.full((N_ZPAD,), N_GRAPHS, jnp.int32)]).reshape(NPAD, 1)
    b1c = b1.reshape(1, D)
    b2c = b2.reshape(1, D)
    blc = bl.reshape(1, D)

    deg_parts = _sc_degree(dst_arr)
    h1 = _tc_matmul(x_pad, W1)
    d0 = deg_parts[0].reshape(NPAD, 1)
    d1 = deg_parts[1].reshape(NPAD, 1)
    p1, dis = _tc_scale(h1, d0, d1)

    acc1 = _sc_edges(p1, src_arr, dst_arr)
    p2 = _tc_mid(acc1[0], acc1[1], p1, dis, b1c, W2)

    acc2 = _sc_edges(p2, src_arr, dst_arr)
    return _tc_pool(acc2[0], acc2[1], p2, dis, b2c, batch_col, Wl, blc)


# speedup vs baseline: 1.0289x; 1.0093x over previous
import functools

import jax
import jax.numpy as jnp
from jax import lax
from jax.experimental import pallas as pl
from jax.experimental.pallas import tpu as pltpu
from jax.experimental.pallas import tpu_sc as plsc

N_NODES = 10000
D = 128
N_GRAPHS = 16

NPAD = 10240
N_ZPAD = NPAD - N_NODES
NC = 2
NS = 16
NW = NC * NS
CHUNK = 128
CPT = 80
PHASES = 2
CPP = CPT // PHASES
NBUF = 2
EPAD = NW * CPT * CHUNK
ROWS_PT = NPAD // NS

_sc_mesh = plsc.VectorSubcoreMesh(core_axis_name="c", subcore_axis_name="s")


def _sc_degree_body(dst_hbm, out_hbm, dst_v, ones_v, zero_v, deg_acc,
                    zsem, dsems):
    c = lax.axis_index("c")
    s = lax.axis_index("s")
    w = c * NS + s

    for i in range(CHUNK // 16):
        ones_v[pl.ds(i * 16, 16)] = jnp.ones((16,), jnp.float32)
        zero_v[pl.ds(i * 16, 16)] = jnp.zeros((16,), jnp.float32)

    for t in range(ROWS_PT // CHUNK):
        pltpu.async_copy(zero_v,
                         deg_acc.at[pl.ds(s * ROWS_PT + t * CHUNK, CHUNK)],
                         zsem)

    pltpu.sync_copy(dst_hbm.at[w], dst_v)

    for t in range(ROWS_PT // CHUNK):
        pltpu.make_async_copy(
            zero_v, deg_acc.at[pl.ds(s * ROWS_PT + t * CHUNK, CHUNK)],
            zsem).wait()

    plsc.subcore_barrier()

    def dscat_start(j, b):
        pltpu.async_copy(ones_v, deg_acc.at[dst_v.at[j]], dsems.at[b],
                         add=True)

    def dscat_wait(j, b):
        pltpu.make_async_copy(ones_v, deg_acc.at[dst_v.at[j]],
                              dsems.at[b]).wait()

    dscat_start(0, 0)
    dscat_start(1, 1)

    def body(i, carry):
        j0 = 2 * i
        j1 = 2 * i + 1
        dscat_wait(j0, 0)

        @pl.when(j0 + 2 < CPT)
        def _():
            dscat_start(j0 + 2, 0)

        dscat_wait(j1, 1)

        @pl.when(j1 + 2 < CPT)
        def _():
            dscat_start(j1 + 2, 1)

        return carry

    lax.fori_loop(0, CPT // 2, body, 0)

    plsc.subcore_barrier()
    pltpu.sync_copy(deg_acc.at[pl.ds(s * ROWS_PT, ROWS_PT)],
                    out_hbm.at[c, pl.ds(s * ROWS_PT, ROWS_PT)])


_sc_degree = pl.kernel(
    _sc_degree_body,
    out_type=jax.ShapeDtypeStruct((NC, NPAD), jnp.float32),
    mesh=_sc_mesh,
    scratch_types=[
        pltpu.VMEM((CPT, CHUNK), jnp.int32),
        pltpu.VMEM((CHUNK,), jnp.float32),
        pltpu.VMEM((CHUNK,), jnp.float32),
        pltpu.VMEM_SHARED((NPAD,), jnp.float32),
        pltpu.SemaphoreType.DMA,
        pltpu.SemaphoreType.DMA((2,)),
    ],
)


def _sc_edges_body_full(p_hbm, src_hbm, dst_hbm, out_hbm,
                        src_v, dst_v, rows, acc, gsems, isem):
    c = lax.axis_index("c")
    s = lax.axis_index("s")
    w = c * NS + s

    init = pltpu.async_copy(p_hbm.at[pl.ds(s * ROWS_PT, ROWS_PT)],
                            acc.at[pl.ds(s * ROWS_PT, ROWS_PT)], isem)

    def gather_start(j, b):
        pltpu.async_copy(p_hbm.at[src_v.at[j]], rows.at[b], gsems.at[b])

    def gather_wait(j, b):
        pltpu.make_async_copy(p_hbm.at[src_v.at[j]], rows.at[b],
                              gsems.at[b]).wait()

    def scatter(j, b):
        pltpu.sync_copy(rows.at[b], acc.at[dst_v.at[j]], add=True)

    for ph in range(PHASES):
        pltpu.sync_copy(src_hbm.at[w, pl.ds(ph * CPP, CPP)], src_v)
        pltpu.sync_copy(dst_hbm.at[w, pl.ds(ph * CPP, CPP)], dst_v)
        gather_start(0, 0)
        if ph == 0:
            init.wait()
            plsc.subcore_barrier()

        def body(i, carry):
            j0 = 2 * i
            j1 = 2 * i + 1
            gather_start(j1, 1)
            gather_wait(j0, 0)
            scatter(j0, 0)

            @pl.when(j1 + 1 < CPP)
            def _():
                gather_start(j1 + 1, 0)

            gather_wait(j1, 1)
            scatter(j1, 1)
            return carry

        lax.fori_loop(0, CPP // 2, body, 0)

    plsc.subcore_barrier()
    pltpu.sync_copy(acc.at[pl.ds(s * ROWS_PT, ROWS_PT)],
                    out_hbm.at[c, pl.ds(s * ROWS_PT, ROWS_PT)])


_sc_edges = pl.kernel(
    _sc_edges_body_full,
    out_type=jax.ShapeDtypeStruct((NC, NPAD, D), jnp.float32),
    mesh=_sc_mesh,
    scratch_types=[
        pltpu.VMEM((CPP, CHUNK), jnp.int32),
        pltpu.VMEM((CPP, CHUNK), jnp.int32),
        pltpu.VMEM((NBUF, CHUNK, D), jnp.float32),
        pltpu.VMEM_SHARED((NPAD, D), jnp.float32),
        pltpu.SemaphoreType.DMA((NBUF,)),
        pltpu.SemaphoreType.DMA,
    ],
)


BLK = 5120
GRID = NPAD // BLK


def _tc_matmul_body(x_ref, w_ref, h_ref):
    h_ref[...] = jnp.dot(x_ref[...], w_ref[...],
                         preferred_element_type=jnp.float32)


def _tc_matmul(x_pad, W1):
    return pl.pallas_call(
        _tc_matmul_body,
        grid=(GRID,),
        in_specs=[
            pl.BlockSpec((BLK, D), lambda i: (i, 0)),
            pl.BlockSpec((D, D), lambda i: (0, 0)),
        ],
        out_specs=pl.BlockSpec((BLK, D), lambda i: (i, 0)),
        out_shape=jax.ShapeDtypeStruct((NPAD, D), jnp.float32),
    )(x_pad, W1)


def _tc_scale_body(h_ref, d0_ref, d1_ref, p_ref, dis_ref):
    deg = d0_ref[...] + d1_ref[...] + 1.0
    dis = lax.rsqrt(deg)
    dis_ref[...] = dis
    p_ref[...] = h_ref[...] * dis


def _tc_scale(h1, d0, d1):
    return pl.pallas_call(
        _tc_scale_body,
        grid=(GRID,),
        in_specs=[
            pl.BlockSpec((BLK, D), lambda i: (i, 0)),
            pl.BlockSpec((BLK, 1), lambda i: (i, 0)),
            pl.BlockSpec((BLK, 1), lambda i: (i, 0)),
        ],
        out_specs=[
            pl.BlockSpec((BLK, D), lambda i: (i, 0)),
            pl.BlockSpec((BLK, 1), lambda i: (i, 0)),
        ],
        out_shape=[
            jax.ShapeDtypeStruct((NPAD, D), jnp.float32),
            jax.ShapeDtypeStruct((NPAD, 1), jnp.float32),
        ],
    )(h1, d0, d1)


def _tc_mid_body(a0_ref, a1_ref, p_ref, dis_ref, b1_ref, w2_ref, out_ref, *, blk):
    i = pl.program_id(0)
    agg = a0_ref[...] + a1_ref[...] - p_ref[...]
    h = jnp.maximum(agg * dis_ref[...] + b1_ref[...], 0.0)
    p2 = jnp.dot(h, w2_ref[...], preferred_element_type=jnp.float32) * dis_ref[...]
    row = i * blk + lax.broadcasted_iota(jnp.int32, (blk, 1), 0)
    out_ref[...] = jnp.where(row < N_NODES, p2, 0.0)


def _tc_mid(a0, a1, p1, dis, b1, W2):
    return pl.pallas_call(
        functools.partial(_tc_mid_body, blk=BLK),
        grid=(GRID,),
        in_specs=[
            pl.BlockSpec((BLK, D), lambda i: (i, 0)),
            pl.BlockSpec((BLK, D), lambda i: (i, 0)),
            pl.BlockSpec((BLK, D), lambda i: (i, 0)),
            pl.BlockSpec((BLK, 1), lambda i: (i, 0)),
            pl.BlockSpec((1, D), lambda i: (0, 0)),
            pl.BlockSpec((D, D), lambda i: (0, 0)),
        ],
        out_specs=pl.BlockSpec((BLK, D), lambda i: (i, 0)),
        out_shape=jax.ShapeDtypeStruct((NPAD, D), jnp.float32),
    )(a0, a1, p1, dis, b1, W2)


def _tc_pool_body(a0_ref, a1_ref, p_ref, dis_ref, b2_ref, batch_ref,
                  wl_ref, bl_ref, out_ref, gsum, cnt, *, blk, grid):
    i = pl.program_id(0)

    @pl.when(i == 0)
    def _():
        gsum[...] = jnp.zeros_like(gsum)
        cnt[...] = jnp.zeros_like(cnt)

    agg = a0_ref[...] + a1_ref[...] - p_ref[...]
    h2 = agg * dis_ref[...] + b2_ref[...]
    gids = lax.broadcasted_iota(jnp.int32, (1, N_GRAPHS), 1)
    oh = (batch_ref[...] == gids).astype(jnp.float32)
    dnums = (((0,), (0,)), ((), ()))
    gsum[...] += lax.dot_general(oh, h2, dnums,
                                 preferred_element_type=jnp.float32)
    cnt[...] += lax.dot_general(oh, jnp.ones((blk, D), jnp.float32), dnums,
                                preferred_element_type=jnp.float32)

    @pl.when(i == grid - 1)
    def _():
        g = gsum[...] / jnp.maximum(cnt[...], 1.0)
        out_ref[...] = jnp.dot(g, wl_ref[...],
                               preferred_element_type=jnp.float32) + bl_ref[...]


def _tc_pool(a0, a1, p2, dis, b2, batch_col, Wl, bl):
    return pl.pallas_call(
        functools.partial(_tc_pool_body, blk=BLK, grid=GRID),
        grid=(GRID,),
        in_specs=[
            pl.BlockSpec((BLK, D), lambda i: (i, 0)),
            pl.BlockSpec((BLK, D), lambda i: (i, 0)),
            pl.BlockSpec((BLK, D), lambda i: (i, 0)),
            pl.BlockSpec((BLK, 1), lambda i: (i, 0)),
            pl.BlockSpec((1, D), lambda i: (0, 0)),
            pl.BlockSpec((BLK, 1), lambda i: (i, 0)),
            pl.BlockSpec((D, D), lambda i: (0, 0)),
            pl.BlockSpec((1, D), lambda i: (0, 0)),
        ],
        out_specs=pl.BlockSpec((N_GRAPHS, D), lambda i: (0, 0)),
        out_shape=jax.ShapeDtypeStruct((N_GRAPHS, D), jnp.float32),
        scratch_shapes=[
            pltpu.VMEM((N_GRAPHS, D), jnp.float32),
            pltpu.VMEM((N_GRAPHS, D), jnp.float32),
        ],
    )(a0, a1, p2, dis, b2, batch_col, Wl, bl)


@jax.jit
def kernel(x, edge_index, batch, W1, b1, W2, b2, Wl, bl):
    x_pad = jnp.pad(x, ((0, N_ZPAD), (0, 0)))
    src = edge_index[0].astype(jnp.int32)
    dst = edge_index[1].astype(jnp.int32)
    n_epad = EPAD - src.shape[0]
    pad_idx = N_NODES + (jnp.arange(n_epad, dtype=jnp.int32) % N_ZPAD)
    src_arr = jnp.concatenate([src, pad_idx]).reshape(NW, CPT, CHUNK)
    dst_arr = jnp.concatenate([dst, pad_idx]).reshape(NW, CPT, CHUNK)
    batch_col = jnp.concatenate(
        [batch.astype(jnp.int32),
         jnp.full((N_ZPAD,), N_GRAPHS, jnp.int32)]).reshape(NPAD, 1)
    b1c = b1.reshape(1, D)
    b2c = b2.reshape(1, D)
    blc = bl.reshape(1, D)

    deg_parts = _sc_degree(dst_arr)
    h1 = _tc_matmul(x_pad, W1)
    d0 = deg_parts[0].reshape(NPAD, 1)
    d1 = deg_parts[1].reshape(NPAD, 1)
    p1, dis = _tc_scale(h1, d0, d1)

    acc1 = _sc_edges(p1, src_arr, dst_arr)
    p2 = _tc_mid(acc1[0], acc1[1], p1, dis, b1c, W2)

    acc2 = _sc_edges(p2, src_arr, dst_arr)
    return _tc_pool(acc2[0], acc2[1], p2, dis, b2c, batch_col, Wl, blc)
